# baseline (speedup 1.0000x reference)
import functools

import jax
import jax.numpy as jnp
from jax import lax
from jax.experimental import pallas as pl
from jax.experimental.pallas import tpu as pltpu
from jax.experimental.pallas import tpu_sc as plsc

N_NODES = 100000
IN_DIM = 16
NON_ZERO = 128
FEATURES_DIM = 200
NUM_CORES = 2
NUM_SUBCORES = 16
NUM_WORKERS = NUM_CORES * NUM_SUBCORES
CHUNK = 200
NUM_CHUNKS = N_NODES // CHUNK
LANES = 16
NUM_GROUPS = NON_ZERO // LANES

_GATHER_DNUMS = lax.GatherDimensionNumbers(
    offset_dims=(), collapsed_slice_dims=(0,), start_index_map=(0,))


def _sc_kernel_body(cd_hbm, ovl_hbm, sel_hbm, out_hbm,
                    in_v, out_v, ovl_v, sel_v, sem):
    wid = lax.axis_index("s") * NUM_CORES + lax.axis_index("c")

    pltpu.sync_copy(ovl_hbm, ovl_v)
    pltpu.sync_copy(sel_hbm, sel_v)

    sel_vecs = [sel_v[pl.ds(LANES * j, LANES)] for j in range(NUM_GROUPS)]
    ovl_vecs = [ovl_v[pl.ds(LANES * j, LANES)] for j in range(NUM_GROUPS)]

    zeros = jnp.zeros((LANES,), jnp.float32)

    def zero_row(n, carry):
        for off in (128, 144, 160, 176, 184):
            out_v[n, pl.ds(off, LANES)] = zeros
        return carry

    lax.fori_loop(0, CHUNK, zero_row, 0)

    my_chunks = (NUM_CHUNKS - 1 - wid) // NUM_WORKERS + 1

    def chunk_body(i, carry):
        c = wid + i * NUM_WORKERS
        row0 = pl.multiple_of(c * CHUNK, CHUNK)
        cp_in = pltpu.make_async_copy(cd_hbm.at[pl.ds(row0, CHUNK)], in_v, sem)
        cp_in.start()
        cp_in.wait()

        def node_body(n, carry2):
            row = in_v[n, :]
            for j in range(NUM_GROUPS):
                g = lax.gather(
                    row, sel_vecs[j][:, None], _GATHER_DNUMS, (1,),
                    mode=lax.GatherScatterMode.PROMISE_IN_BOUNDS)
                out_v[n, pl.ds(LANES * j, LANES)] = g * ovl_vecs[j]
            return carry2

        lax.fori_loop(0, CHUNK, node_body, 0)

        cp_out = pltpu.make_async_copy(out_v, out_hbm.at[pl.ds(row0, CHUNK)], sem)
        cp_out.start()
        cp_out.wait()
        return carry

    lax.fori_loop(0, my_chunks, chunk_body, 0)


def kernel(charge_density, overlap_constants, select_indices):
    sel32 = select_indices.astype(jnp.int32)
    mesh = plsc.VectorSubcoreMesh(core_axis_name="c", subcore_axis_name="s")
    run = functools.partial(
        pl.kernel,
        mesh=mesh,
        out_type=jax.ShapeDtypeStruct((N_NODES, FEATURES_DIM), jnp.float32),
        scratch_types=[
            pltpu.VMEM((CHUNK, IN_DIM), jnp.float32),
            pltpu.VMEM((CHUNK, FEATURES_DIM), jnp.float32),
            pltpu.VMEM((NON_ZERO,), jnp.float32),
            pltpu.VMEM((NON_ZERO,), jnp.int32),
            pltpu.SemaphoreType.DMA,
        ],
    )(_sc_kernel_body)
    return run(charge_density, overlap_constants, sel32)


# speedup vs baseline: 1.2048x; 1.2048x over previous
import functools

import jax
import jax.numpy as jnp
from jax import lax
from jax.experimental import pallas as pl
from jax.experimental.pallas import tpu as pltpu
from jax.experimental.pallas import tpu_sc as plsc

N_NODES = 100000
IN_DIM = 16
NON_ZERO = 128
FEATURES_DIM = 200
NUM_CORES = 2
NUM_SUBCORES = 16
NUM_WORKERS = NUM_CORES * NUM_SUBCORES
CHUNK = 160
NUM_CHUNKS = N_NODES // CHUNK
MAX_PAIRS = ((NUM_CHUNKS + NUM_WORKERS - 1) // NUM_WORKERS + 1) // 2
LANES = 16
NUM_GROUPS = NON_ZERO // LANES
NODE_UNROLL = 2

_GATHER_DNUMS = lax.GatherDimensionNumbers(
    offset_dims=(), collapsed_slice_dims=(0,), start_index_map=(0,))


def _sc_kernel_body(cd_hbm, ovl_hbm, sel_hbm, out_hbm,
                    in0, in1, out0, out1, ovl_v, sel_v,
                    sem_in0, sem_in1, sem_out0, sem_out1):
    wid = lax.axis_index("s") * NUM_CORES + lax.axis_index("c")

    pltpu.sync_copy(ovl_hbm, ovl_v)
    pltpu.sync_copy(sel_hbm, sel_v)

    sel_vecs = [sel_v[pl.ds(LANES * j, LANES)] for j in range(NUM_GROUPS)]
    ovl_vecs = [ovl_v[pl.ds(LANES * j, LANES)] for j in range(NUM_GROUPS)]

    in_bufs = (in0, in1)
    out_bufs = (out0, out1)
    sem_ins = (sem_in0, sem_in1)
    sem_outs = (sem_out0, sem_out1)

    zeros = jnp.zeros((LANES,), jnp.float32)

    def zero_row(n, carry):
        for ob in out_bufs:
            for off in (128, 144, 160, 176, 184):
                ob[n, pl.ds(off, LANES)] = zeros
        return carry

    lax.fori_loop(0, CHUNK, zero_row, 0)

    my_chunks = (NUM_CHUNKS - 1 - wid) // NUM_WORKERS + 1

    def row_of(j):
        return pl.multiple_of((wid + j * NUM_WORKERS) * CHUNK, CHUNK)

    def in_copy(j, b):
        return pltpu.make_async_copy(
            cd_hbm.at[pl.ds(row_of(j), CHUNK)], in_bufs[b], sem_ins[b])

    def out_copy(j, b):
        return pltpu.make_async_copy(
            out_bufs[b], out_hbm.at[pl.ds(row_of(j), CHUNK)], sem_outs[b])

    @pl.when(my_chunks > 0)
    def _():
        in_copy(0, 0).start()

    def pair_body(i, carry):
        for b in range(2):
            j = i * 2 + b

            @pl.when(j < my_chunks)
            def _():
                @pl.when(j + 1 < my_chunks)
                def _():
                    in_copy(j + 1, 1 - b).start()

                in_copy(j, b).wait()

                @pl.when(j >= 2)
                def _():
                    out_copy(j, b).wait()

                def node_body(n2, carry2):
                    for d in range(NODE_UNROLL):
                        n = n2 * NODE_UNROLL + d
                        row = in_bufs[b][n, :]
                        for g in range(NUM_GROUPS):
                            v = lax.gather(
                                row, sel_vecs[g][:, None], _GATHER_DNUMS,
                                (1,),
                                mode=lax.GatherScatterMode.PROMISE_IN_BOUNDS)
                            out_bufs[b][n, pl.ds(LANES * g, LANES)] = (
                                v * ovl_vecs[g])
                    return carry2

                lax.fori_loop(0, CHUNK // NODE_UNROLL, node_body, 0)

                out_copy(j, b).start()
        return carry

    lax.fori_loop(0, MAX_PAIRS, pair_body, 0)

    for b in range(2):
        @pl.when(my_chunks >= b + 1)
        def _():
            out_copy(b, b).wait()


def kernel(charge_density, overlap_constants, select_indices):
    sel32 = select_indices.astype(jnp.int32)
    mesh = plsc.VectorSubcoreMesh(core_axis_name="c", subcore_axis_name="s")
    run = functools.partial(
        pl.kernel,
        mesh=mesh,
        out_type=jax.ShapeDtypeStruct((N_NODES, FEATURES_DIM), jnp.float32),
        scratch_types=[
            pltpu.VMEM((CHUNK, IN_DIM), jnp.float32),
            pltpu.VMEM((CHUNK, IN_DIM), jnp.float32),
            pltpu.VMEM((CHUNK, FEATURES_DIM), jnp.float32),
            pltpu.VMEM((CHUNK, FEATURES_DIM), jnp.float32),
            pltpu.VMEM((NON_ZERO,), jnp.float32),
            pltpu.VMEM((NON_ZERO,), jnp.int32),
            pltpu.SemaphoreType.DMA,
            pltpu.SemaphoreType.DMA,
            pltpu.SemaphoreType.DMA,
            pltpu.SemaphoreType.DMA,
        ],
    )(_sc_kernel_body)
    return run(charge_density, overlap_constants, sel32)


# speedup vs baseline: 3.8580x; 3.2023x over previous
import functools

import jax
import jax.numpy as jnp
from jax import lax
from jax.experimental import pallas as pl
from jax.experimental.pallas import tpu as pltpu
from jax.experimental.pallas import tpu_sc as plsc

N_NODES = 100000
IN_DIM = 16
NON_ZERO = 128
FEATURES_DIM = 200
NUM_CORES = 2
NUM_SUBCORES = 16
NUM_WORKERS = NUM_CORES * NUM_SUBCORES
LANES = 16

CW = 128
NUM_FULL = N_NODES // CW
REM = N_NODES - NUM_FULL * CW
REM_OFF = NUM_FULL * CW
REM_WORKER = 30
VPC = CW // LANES
VPC_REM = REM // LANES
MAX_PAIRS = ((NUM_FULL + NUM_WORKERS - 1) // NUM_WORKERS + 1) // 2

_SRC = [ll * ll + m
        for ll in range(4) for _s in range(8) for m in range(2 * ll + 1)]
assert len(_SRC) == NON_ZERO

_GATHER_DNUMS = lax.GatherDimensionNumbers(
    offset_dims=(), collapsed_slice_dims=(0,), start_index_map=(0,))


def _splat(vec, lane):
    idx = jnp.full((LANES,), lane, jnp.int32)
    return lax.gather(vec, idx[:, None], _GATHER_DNUMS, (1,),
                      mode=lax.GatherScatterMode.PROMISE_IN_BOUNDS)


def _sc_kernel_body(cd_hbm, ovl_hbm, out_hbm,
                    in0, in1, out0, out1, in_r, out_r, ovl_v,
                    sem_in0, sem_in1, sem_out0, sem_out1):
    wid = lax.axis_index("s") * NUM_CORES + lax.axis_index("c")

    pltpu.sync_copy(ovl_hbm, ovl_v)
    ovl_vecs = [ovl_v[pl.ds(LANES * g, LANES)] for g in range(NON_ZERO // LANES)]

    in_bufs = (in0, in1)
    out_bufs = (out0, out1)
    sem_ins = (sem_in0, sem_in1)
    sem_outs = (sem_out0, sem_out1)

    zeros = jnp.zeros((LANES,), jnp.float32)

    def zero_row(r, carry):
        for ob in out_bufs:
            for v in range(VPC):
                ob[NON_ZERO + r, pl.ds(LANES * v, LANES)] = zeros
        for v in range(VPC_REM):
            out_r[NON_ZERO + r, pl.ds(LANES * v, LANES)] = zeros
        return carry

    lax.fori_loop(0, FEATURES_DIM - NON_ZERO, zero_row, 0)

    def compute_cols(in_ref, out_ref, n_vregs):
        def col_body(v, carry):
            off = pl.ds(LANES * v, LANES)
            for src in range(IN_DIM):
                row = in_ref[src, off]
                for j, s in enumerate(_SRC):
                    if s != src:
                        continue
                    scale = _splat(ovl_vecs[j // LANES], j % LANES)
                    out_ref[j, off] = row * scale
            return carry

        lax.fori_loop(0, n_vregs, col_body, 0)

    @pl.when(wid == REM_WORKER)
    def _():
        rem = pl.ds(REM_OFF, REM)
        cp = pltpu.make_async_copy(cd_hbm.at[:, rem], in_r, sem_in0)
        cp.start()
        cp.wait()
        compute_cols(in_r, out_r, VPC_REM)
        cp2 = pltpu.make_async_copy(out_r, out_hbm.at[:, rem], sem_out0)
        cp2.start()
        cp2.wait()

    my_chunks = (NUM_FULL - 1 - wid) // NUM_WORKERS + 1

    def col_of(j):
        return pl.multiple_of((wid + j * NUM_WORKERS) * CW, CW)

    def in_copy(j, b):
        return pltpu.make_async_copy(
            cd_hbm.at[:, pl.ds(col_of(j), CW)], in_bufs[b], sem_ins[b])

    def out_copy(j, b):
        return pltpu.make_async_copy(
            out_bufs[b], out_hbm.at[:, pl.ds(col_of(j), CW)], sem_outs[b])

    in_copy(0, 0).start()

    def pair_body(i, carry):
        for b in range(2):
            j = i * 2 + b

            @pl.when(j < my_chunks)
            def _():
                @pl.when(j + 1 < my_chunks)
                def _():
                    in_copy(j + 1, 1 - b).start()

                in_copy(j, b).wait()

                @pl.when(j >= 2)
                def _():
                    out_copy(j, b).wait()

                compute_cols(in_bufs[b], out_bufs[b], VPC)
                out_copy(j, b).start()
        return carry

    lax.fori_loop(0, MAX_PAIRS, pair_body, 0)

    for b in range(2):
        @pl.when(my_chunks >= b + 1)
        def _():
            out_copy(b, b).wait()


def kernel(charge_density, overlap_constants, select_indices):
    del select_indices
    cd_t = charge_density.T
    mesh = plsc.VectorSubcoreMesh(core_axis_name="c", subcore_axis_name="s")
    run = functools.partial(
        pl.kernel,
        mesh=mesh,
        out_type=jax.ShapeDtypeStruct((FEATURES_DIM, N_NODES), jnp.float32),
        scratch_types=[
            pltpu.VMEM((IN_DIM, CW), jnp.float32),
            pltpu.VMEM((IN_DIM, CW), jnp.float32),
            pltpu.VMEM((FEATURES_DIM, CW), jnp.float32),
            pltpu.VMEM((FEATURES_DIM, CW), jnp.float32),
            pltpu.VMEM((IN_DIM, REM), jnp.float32),
            pltpu.VMEM((FEATURES_DIM, REM), jnp.float32),
            pltpu.VMEM((NON_ZERO,), jnp.float32),
            pltpu.SemaphoreType.DMA,
            pltpu.SemaphoreType.DMA,
            pltpu.SemaphoreType.DMA,
            pltpu.SemaphoreType.DMA,
        ],
    )(_sc_kernel_body)
    out_t = run(cd_t, overlap_constants)
    return out_t.T
